# baseline (speedup 1.0000x reference)
import functools

import jax
import jax.numpy as jnp
from jax.experimental import pallas as pl
from jax.experimental.pallas import tpu as pltpu


def _round_up(x: int, m: int) -> int:
    return ((x + m - 1) // m) * m


def _se_kernel(x_any, w1t_ref, w2t_ref, o_any, buf, in_sem, out_sem,
               *, inv_hw, nb):
    cp_in = pltpu.make_async_copy(x_any, buf, in_sem)
    cp_in.start()
    cp_in.wait()
    for j in range(nb):
        xb = buf[j].astype(jnp.float32)
        pooled = jnp.sum(xb, axis=-1, keepdims=True) * inv_hw
        h = jnp.dot(w1t_ref[...], pooled,
                    preferred_element_type=jnp.float32)
        h = jnp.maximum(h, 0.0)
        z = jnp.dot(w2t_ref[...], h,
                    preferred_element_type=jnp.float32)
        g = jax.nn.sigmoid(z)
        buf[j] = (xb * g).astype(buf.dtype)
    cp_out = pltpu.make_async_copy(buf, o_any, out_sem)
    cp_out.start()
    cp_out.wait()


def kernel(x, w1, w2):
    B, C, H, W = x.shape
    HW = H * W
    rd = w1.shape[1]

    c_pad = _round_up(C, 8)
    hw_pad = _round_up(HW, 128)
    rd_pad = _round_up(rd, 8)

    x3 = x.reshape(B, C, HW)
    if c_pad != C or hw_pad != HW:
        x3 = jnp.pad(x3, ((0, 0), (0, c_pad - C), (0, hw_pad - HW)))

    w1t = w1.astype(jnp.float32).T
    w2t = w2.astype(jnp.float32).T
    if c_pad != C or rd_pad != rd:
        w1t = jnp.pad(w1t, ((0, rd_pad - rd), (0, c_pad - C)))
        w2t = jnp.pad(w2t, ((0, c_pad - C), (0, rd_pad - rd)))

    chunk_bytes = c_pad * hw_pad * x.dtype.itemsize
    nb = B
    while nb > 1 and nb * chunk_bytes > 8 * 1024 * 1024:
        nb //= 2
    ng = (B + nb - 1) // nb

    call = pl.pallas_call(
        functools.partial(_se_kernel, inv_hw=1.0 / HW, nb=nb),
        out_shape=jax.ShapeDtypeStruct((nb, c_pad, hw_pad), x.dtype),
        in_specs=[
            pl.BlockSpec(memory_space=pl.ANY),
            pl.BlockSpec(memory_space=pltpu.MemorySpace.VMEM),
            pl.BlockSpec(memory_space=pltpu.MemorySpace.VMEM),
        ],
        out_specs=pl.BlockSpec(memory_space=pl.ANY),
        scratch_shapes=[
            pltpu.VMEM((nb, c_pad, hw_pad), x.dtype),
            pltpu.SemaphoreType.DMA,
            pltpu.SemaphoreType.DMA,
        ],
        compiler_params=pltpu.CompilerParams(
            vmem_limit_bytes=24 * 1024 * 1024,
        ),
    )

    pieces = []
    for gi in range(ng):
        lo = gi * nb
        xg = x3[lo:lo + nb]
        if xg.shape[0] < nb:
            xg = jnp.pad(xg, ((0, nb - xg.shape[0]), (0, 0), (0, 0)))
        pieces.append(call(xg, w1t, w2t))
    out = jnp.concatenate(pieces, axis=0)[:B]

    if c_pad != C or hw_pad != HW:
        out = out[:, :C, :HW]
    return out.reshape(B, C, H, W)


# speedup vs baseline: 1.7922x; 1.7922x over previous
import functools

import jax
import jax.numpy as jnp
from jax.experimental import pallas as pl
from jax.experimental.pallas import tpu as pltpu


def _round_up(x: int, m: int) -> int:
    return ((x + m - 1) // m) * m


def _se_kernel(x_hbm, w1t_ref, w2t_ref, o_hbm,
               in_buf, out_buf, in_sems, out_sems, *, inv_hw):
    n_chunks = x_hbm.shape[0]
    nin = in_buf.shape[0]
    nout = out_buf.shape[0]

    for k in range(min(nin, n_chunks)):
        pltpu.make_async_copy(x_hbm.at[k], in_buf.at[k], in_sems.at[k]).start()

    for k in range(n_chunks):
        slot = k % nin
        oslot = k % nout
        pltpu.make_async_copy(
            x_hbm.at[k], in_buf.at[slot], in_sems.at[slot]).wait()
        if k >= nout:
            pltpu.make_async_copy(
                out_buf.at[oslot], o_hbm.at[k - nout],
                out_sems.at[oslot]).wait()

        xb = in_buf[slot].astype(jnp.float32)
        pooled = jnp.sum(xb, axis=-1, keepdims=True) * inv_hw
        h = jnp.dot(w1t_ref[...], pooled,
                    preferred_element_type=jnp.float32)
        h = jnp.maximum(h, 0.0)
        z = jnp.dot(w2t_ref[...], h,
                    preferred_element_type=jnp.float32)
        g = jax.nn.sigmoid(z)
        out_buf[oslot] = (xb * g).astype(out_buf.dtype)

        pltpu.make_async_copy(
            out_buf.at[oslot], o_hbm.at[k], out_sems.at[oslot]).start(priority=1)
        if k + nin < n_chunks:
            pltpu.make_async_copy(
                x_hbm.at[k + nin], in_buf.at[slot], in_sems.at[slot]).start()

    for k in range(max(0, n_chunks - nout), n_chunks):
        oslot = k % nout
        pltpu.make_async_copy(
            out_buf.at[oslot], o_hbm.at[k], out_sems.at[oslot]).wait()


def kernel(x, w1, w2):
    B, C, H, W = x.shape
    HW = H * W
    rd = w1.shape[1]

    c_pad = _round_up(C, 8)
    hw_pad = _round_up(HW, 128)
    rd_pad = _round_up(rd, 8)

    x3 = x.reshape(B, C, HW)
    if c_pad != C or hw_pad != HW:
        x3 = jnp.pad(x3, ((0, 0), (0, c_pad - C), (0, hw_pad - HW)))

    w1t = w1.astype(jnp.float32).T
    w2t = w2.astype(jnp.float32).T
    if c_pad != C or rd_pad != rd:
        w1t = jnp.pad(w1t, ((0, rd_pad - rd), (0, c_pad - C)))
        w2t = jnp.pad(w2t, ((0, c_pad - C), (0, rd_pad - rd)))

    chunk_bytes = c_pad * hw_pad * x.dtype.itemsize
    nbuf = max(2, min(8, B, (24 * 1024 * 1024) // chunk_bytes))

    out = pl.pallas_call(
        functools.partial(_se_kernel, inv_hw=1.0 / HW),
        out_shape=jax.ShapeDtypeStruct((B, c_pad, hw_pad), x.dtype),
        in_specs=[
            pl.BlockSpec(memory_space=pl.ANY),
            pl.BlockSpec(memory_space=pltpu.MemorySpace.VMEM),
            pl.BlockSpec(memory_space=pltpu.MemorySpace.VMEM),
        ],
        out_specs=pl.BlockSpec(memory_space=pl.ANY),
        scratch_shapes=[
            pltpu.VMEM((nbuf, c_pad, hw_pad), x.dtype),
            pltpu.VMEM((nbuf, c_pad, hw_pad), x.dtype),
            pltpu.SemaphoreType.DMA((nbuf,)),
            pltpu.SemaphoreType.DMA((nbuf,)),
        ],
        compiler_params=pltpu.CompilerParams(
            vmem_limit_bytes=60 * 1024 * 1024,
        ),
    )(x3, w1t, w2t)

    if c_pad != C or hw_pad != HW:
        out = out[:, :C, :HW]
    return out.reshape(B, C, H, W)


# speedup vs baseline: 1.7941x; 1.0011x over previous
import functools

import jax
import jax.numpy as jnp
from jax.experimental import pallas as pl
from jax.experimental.pallas import tpu as pltpu


def _round_up(x: int, m: int) -> int:
    return ((x + m - 1) // m) * m


def _se_kernel(x_ref, w1t_ref, w2t_ref, o_ref, *, inv_hw, nb):
    for j in range(nb):
        xb = x_ref[j].astype(jnp.float32)
        pooled = jnp.sum(xb, axis=-1, keepdims=True) * inv_hw
        h = jnp.dot(w1t_ref[...], pooled,
                    preferred_element_type=jnp.float32)
        h = jnp.maximum(h, 0.0)
        z = jnp.dot(w2t_ref[...], h,
                    preferred_element_type=jnp.float32)
        g = jax.nn.sigmoid(z)
        o_ref[j] = (xb * g).astype(o_ref.dtype)


def kernel(x, w1, w2):
    B, C, H, W = x.shape
    HW = H * W
    rd = w1.shape[1]

    c_pad = _round_up(C, 8)
    hw_pad = _round_up(HW, 128)
    rd_pad = _round_up(rd, 8)

    x3 = x.reshape(B, C, HW)
    if c_pad != C or hw_pad != HW:
        x3 = jnp.pad(x3, ((0, 0), (0, c_pad - C), (0, hw_pad - HW)))

    w1t = w1.astype(jnp.float32).T
    w2t = w2.astype(jnp.float32).T
    if c_pad != C or rd_pad != rd:
        w1t = jnp.pad(w1t, ((0, rd_pad - rd), (0, c_pad - C)))
        w2t = jnp.pad(w2t, ((0, c_pad - C), (0, rd_pad - rd)))

    nb = 1
    for cand in (4, 2):
        if B % cand == 0 and cand * c_pad * hw_pad * x.dtype.itemsize <= 8 * 1024 * 1024:
            nb = cand
            break

    out = pl.pallas_call(
        functools.partial(_se_kernel, inv_hw=1.0 / HW, nb=nb),
        out_shape=jax.ShapeDtypeStruct((B, c_pad, hw_pad), x.dtype),
        grid=(B // nb,),
        in_specs=[
            pl.BlockSpec((nb, c_pad, hw_pad), lambda b: (b, 0, 0)),
            pl.BlockSpec((rd_pad, c_pad), lambda b: (0, 0)),
            pl.BlockSpec((c_pad, rd_pad), lambda b: (0, 0)),
        ],
        out_specs=pl.BlockSpec((nb, c_pad, hw_pad), lambda b: (b, 0, 0)),
        compiler_params=pltpu.CompilerParams(
            dimension_semantics=("parallel",),
            vmem_limit_bytes=64 * 1024 * 1024,
        ),
    )(x3, w1t, w2t)

    if c_pad != C or hw_pad != HW:
        out = out[:, :C, :HW]
    return out.reshape(B, C, H, W)
